# Initial kernel scaffold; baseline (speedup 1.0000x reference)
import jax
import jax.numpy as jnp
from jax.experimental import pallas as pl


def kernel(logits):
    raise NotImplementedError("write your pallas kernel here")


# speedup vs baseline: 164.2938x; 164.2938x over previous
import functools

import jax
import jax.numpy as jnp
from jax import lax
from jax.experimental import pallas as pl
from jax.experimental.pallas import tpu as pltpu
from jax.experimental.pallas import tpu_sc as plsc

B = 64
V = 100000
L = 16
NB = 512
RANGE = 16.0
INV_DELTA = NB / RANGE
TOP_P = 0.95
NW = 32
ROWS_PER_W = B // NW
VECS = V // L
UNROLL = 10
STEPS = VECS // UNROLL
ZSTEP = (NB * L // L) // 8


def _lane_reduce(vec, op):
    idx = lax.iota(jnp.int32, L)
    for sh in (8, 4, 2, 1):
        perm = jnp.bitwise_xor(idx, sh)
        vec = op(vec, vec.at[perm].get(mode="promise_in_bounds"))
    return vec


def _sc_kernel(logits_hbm, he_hbm, ht_hbm, idx_hbm, row_v, he_v, ht_v,
               si_v):
    wid = lax.axis_index("s") * 2 + lax.axis_index("c")
    iota = lax.iota(jnp.int32, L)
    iota_nb = iota * NB
    zeros = jnp.zeros((L,), jnp.float32)

    for rr in range(ROWS_PER_W):
        r = wid * ROWS_PER_W + rr
        pltpu.sync_copy(logits_hbm.at[r], row_v)

        def zero_body(j, carry):
            for u in range(8):
                he_v[pl.ds((j * 8 + u) * L, L)] = zeros
                ht_v[pl.ds((j * 8 + u) * L, L)] = zeros
            return carry
        lax.fori_loop(0, ZSTEP, zero_body, 0)

        def max_body(i, carry):
            acc_v, acc_i = carry
            base = i * (UNROLL * L)
            for u in range(UNROLL):
                x = row_v[pl.ds(base + u * L, L)]
                gi = iota + (base + u * L)
                upd = x > acc_v
                acc_i = jnp.where(upd, gi, acc_i)
                acc_v = jnp.where(upd, x, acc_v)
            return acc_v, acc_i

        acc_v0 = jnp.full((L,), -jnp.inf, jnp.float32)
        acc_i0 = jnp.zeros((L,), jnp.int32)
        acc_v, acc_i = lax.fori_loop(0, STEPS, max_body, (acc_v0, acc_i0))
        mv = _lane_reduce(acc_v, jnp.maximum)
        cand = jnp.where(acc_v == mv, acc_i, jnp.int32(2**31 - 1))
        gvec = _lane_reduce(cand, jnp.minimum)

        def hist_body(i, carry):
            base = i * (UNROLL * L)
            for u in range(UNROLL):
                x = row_v[pl.ds(base + u * L, L)]
                y = x - mv
                e = jnp.exp(y)
                f = jnp.minimum((mv - x) * INV_DELTA, float(NB - 1))
                bb = jnp.minimum(f.astype(jnp.int32), NB - 1)
                flat = bb + iota_nb
                plsc.addupdate_scatter(he_v, [flat], e)
                plsc.addupdate_scatter(ht_v, [flat], e * y)
            return carry
        lax.fori_loop(0, STEPS, hist_body, 0)

        pltpu.sync_copy(he_v, he_hbm.at[r])
        pltpu.sync_copy(ht_v, ht_hbm.at[r])
        si_v[...] = gvec
        pltpu.sync_copy(si_v, idx_hbm.at[r])


_sc_call = functools.partial(
    pl.kernel,
    out_type=[
        jax.ShapeDtypeStruct((B, L * NB), jnp.float32),
        jax.ShapeDtypeStruct((B, L * NB), jnp.float32),
        jax.ShapeDtypeStruct((B, L), jnp.int32),
    ],
    mesh=plsc.VectorSubcoreMesh(core_axis_name="c", subcore_axis_name="s"),
    compiler_params=pltpu.CompilerParams(needs_layout_passes=False),
    scratch_types=[
        pltpu.VMEM((V,), jnp.float32),
        pltpu.VMEM((L * NB,), jnp.float32),
        pltpu.VMEM((L * NB,), jnp.float32),
        pltpu.VMEM((L,), jnp.int32),
    ],
)(_sc_kernel)


def _finisher(he_ref, ht_ref, conf_ref):
    heb = he_ref[:, 0:NB]
    htb = ht_ref[:, 0:NB]
    for l in range(1, L):
        heb = heb + he_ref[:, l * NB:(l + 1) * NB]
        htb = htb + ht_ref[:, l * NB:(l + 1) * NB]
    z = jnp.sum(heb, axis=-1, keepdims=True)
    rix = lax.broadcasted_iota(jnp.int32, (NB, NB), 0)
    cix = lax.broadcasted_iota(jnp.int32, (NB, NB), 1)
    tri = (rix < cix).astype(jnp.float32)
    cumex = jnp.dot(heb, tri, preferred_element_type=jnp.float32)
    kept = cumex <= TOP_P * z
    s = jnp.sum(jnp.where(kept, heb, 0.0), axis=-1, keepdims=True)
    t = jnp.sum(jnp.where(kept, htb, 0.0), axis=-1, keepdims=True)
    conf_ref[...] = t / s - jnp.log(s)


def kernel(logits):
    assert logits.shape == (B, V) and logits.dtype == jnp.float32
    he, ht, idx = _sc_call(logits)
    conf2 = pl.pallas_call(
        _finisher,
        out_shape=jax.ShapeDtypeStruct((B, 1), jnp.float32),
    )(he, ht)
    return conf2.reshape(B), idx[:, 0]


# speedup vs baseline: 425.7235x; 2.5912x over previous
import functools

import jax
import jax.numpy as jnp
from jax import lax
from jax.experimental import pallas as pl
from jax.experimental.pallas import tpu as pltpu
from jax.experimental.pallas import tpu_sc as plsc

B = 64
V = 100000
L = 16
NB = 512
RANGE = 16.0
INV_DELTA = NB / RANGE
TOP_P = 0.95
NW = 32
ROWS_PER_W = B // NW
VECS = V // L
UNROLL = 10
STEPS = VECS // UNROLL
ZSTEP = (NB * L // L) // 8


def _lane_reduce(vec, op):
    idx = lax.iota(jnp.int32, L)
    for sh in (8, 4, 2, 1):
        perm = jnp.bitwise_xor(idx, sh)
        vec = op(vec, vec.at[perm].get(mode="promise_in_bounds"))
    return vec


def _sc_kernel(logits_hbm, he_hbm, ht_hbm, idx_hbm, row_v, he_v, ht_v,
               si_v):
    wid = lax.axis_index("s") * 2 + lax.axis_index("c")
    iota = lax.iota(jnp.int32, L)
    iota_nb = iota * NB
    zeros = jnp.zeros((L,), jnp.float32)

    for rr in range(ROWS_PER_W):
        r = wid * ROWS_PER_W + rr
        pltpu.sync_copy(logits_hbm.at[r], row_v)

        @plsc.parallel_loop(0, NB * L // L, unroll=8)
        def _(j):
            he_v[pl.ds(j * L, L)] = zeros
            ht_v[pl.ds(j * L, L)] = zeros

        acc_v0 = jnp.full((L,), -jnp.inf, jnp.float32)
        acc_i0 = jnp.zeros((L,), jnp.int32)

        @plsc.parallel_loop(0, VECS, unroll=8, carry=(acc_v0, acc_i0))
        def max_loop(i, carry):
            acc_v, acc_i = carry
            x = row_v[pl.ds(i * L, L)]
            gi = iota + i * L
            upd = x > acc_v
            acc_i = jnp.where(upd, gi, acc_i)
            acc_v = jnp.where(upd, x, acc_v)
            return acc_v, acc_i

        acc_v, acc_i = max_loop
        mv = _lane_reduce(acc_v, jnp.maximum)
        cand = jnp.where(acc_v == mv, acc_i, jnp.int32(2**31 - 1))
        gvec = _lane_reduce(cand, jnp.minimum)

        @plsc.parallel_loop(0, VECS, unroll=8)
        def _(i):
            x = row_v[pl.ds(i * L, L)]
            y = x - mv
            e = jnp.exp(y)
            f = jnp.minimum((mv - x) * INV_DELTA, float(NB - 1))
            flat = f.astype(jnp.int32) + iota_nb
            plsc.addupdate_scatter(he_v, [flat], e)
            plsc.addupdate_scatter(ht_v, [flat], e * y)

        pltpu.sync_copy(he_v, he_hbm.at[r])
        pltpu.sync_copy(ht_v, ht_hbm.at[r])
        si_v[...] = gvec
        pltpu.sync_copy(si_v, idx_hbm.at[r])


_sc_call = functools.partial(
    pl.kernel,
    out_type=[
        jax.ShapeDtypeStruct((B, L * NB), jnp.float32),
        jax.ShapeDtypeStruct((B, L * NB), jnp.float32),
        jax.ShapeDtypeStruct((B, L), jnp.int32),
    ],
    mesh=plsc.VectorSubcoreMesh(core_axis_name="c", subcore_axis_name="s"),
    compiler_params=pltpu.CompilerParams(needs_layout_passes=False),
    scratch_types=[
        pltpu.VMEM((V,), jnp.float32),
        pltpu.VMEM((L * NB,), jnp.float32),
        pltpu.VMEM((L * NB,), jnp.float32),
        pltpu.VMEM((L,), jnp.int32),
    ],
)(_sc_kernel)


def _finisher(he_ref, ht_ref, conf_ref):
    heb = he_ref[:, 0:NB]
    htb = ht_ref[:, 0:NB]
    for l in range(1, L):
        heb = heb + he_ref[:, l * NB:(l + 1) * NB]
        htb = htb + ht_ref[:, l * NB:(l + 1) * NB]
    z = jnp.sum(heb, axis=-1, keepdims=True)
    rix = lax.broadcasted_iota(jnp.int32, (NB, NB), 0)
    cix = lax.broadcasted_iota(jnp.int32, (NB, NB), 1)
    tri = (rix < cix).astype(jnp.float32)
    cumex = jnp.dot(heb, tri, preferred_element_type=jnp.float32)
    kept = cumex <= TOP_P * z
    s = jnp.sum(jnp.where(kept, heb, 0.0), axis=-1, keepdims=True)
    t = jnp.sum(jnp.where(kept, htb, 0.0), axis=-1, keepdims=True)
    conf_ref[...] = t / s - jnp.log(s)


def kernel(logits):
    assert logits.shape == (B, V) and logits.dtype == jnp.float32
    he, ht, idx = _sc_call(logits)
    conf2 = pl.pallas_call(
        _finisher,
        out_shape=jax.ShapeDtypeStruct((B, 1), jnp.float32),
    )(he, ht)
    return conf2.reshape(B), idx[:, 0]
